# baseline (speedup 1.0000x reference)
import jax
import jax.numpy as jnp
from jax.experimental import pallas as pl
from jax.experimental.pallas import tpu as pltpu

BATCH = 16384
NUM_FEATURES = 26
EMB_DIM = 64
BLOCK_B = 1024
NCHUNKS = 4
CHUNK_B = BATCH // NCHUNKS
NB_CHUNK = CHUNK_B // BLOCK_B


def _tc_body(x_ref, w_ref, out_ref, norm_ref):
    w = w_ref[...]
    e = jnp.exp(w)
    s = jnp.sum(e, axis=0, keepdims=True)
    n = e / s
    norm_ref[...] = n
    x = x_ref[...]
    out_ref[...] = jnp.sum(x * n[:, None, :], axis=0)


def _tc_body_acc(x_ref, w_ref, oin_ref, nin_ref, out_ref, norm_ref):
    del oin_ref, nin_ref
    _tc_body(x_ref, w_ref, out_ref, norm_ref)


_OUT_SHAPES = (
    jax.ShapeDtypeStruct((EMB_DIM, BATCH), jnp.float32),
    jax.ShapeDtypeStruct((NUM_FEATURES, BATCH), jnp.float32),
)


def _pool_chunk(xt, g_t, chunk, carry):
    off = chunk * NB_CHUNK
    in_specs = [
        pl.BlockSpec((NUM_FEATURES, EMB_DIM, BLOCK_B),
                     lambda i, off=off: (0, 0, off + i)),
        pl.BlockSpec((NUM_FEATURES, BLOCK_B), lambda i: (0, i)),
    ]
    out_specs = (
        pl.BlockSpec((EMB_DIM, BLOCK_B), lambda i, off=off: (0, off + i)),
        pl.BlockSpec((NUM_FEATURES, BLOCK_B), lambda i, off=off: (0, off + i)),
    )
    if carry is None:
        return pl.pallas_call(
            _tc_body,
            grid=(NB_CHUNK,),
            in_specs=in_specs,
            out_specs=out_specs,
            out_shape=_OUT_SHAPES,
        )(xt, g_t)
    in_specs += [
        pl.BlockSpec(memory_space=pl.ANY),
        pl.BlockSpec(memory_space=pl.ANY),
    ]
    return pl.pallas_call(
        _tc_body_acc,
        grid=(NB_CHUNK,),
        in_specs=in_specs,
        out_specs=out_specs,
        out_shape=_OUT_SHAPES,
        input_output_aliases={2: 0, 3: 1},
    )(xt, g_t, carry[0], carry[1])


@jax.jit
def kernel(inputs, item_indices, attention_weights):
    xt = jnp.transpose(inputs, (1, 2, 0))
    gts = []
    for c in range(NCHUNKS):
        idx_c = jax.lax.slice(item_indices, (c * CHUNK_B,), ((c + 1) * CHUNK_B,))
        gts.append(jnp.take(attention_weights, idx_c, axis=0).T)
    carry = None
    for c in range(NCHUNKS):
        carry = _pool_chunk(xt, gts[c], c, carry)
    out_t, norm_t = carry
    return out_t.T, norm_t.T[:, :, None]


# speedup vs baseline: 1.1811x; 1.1811x over previous
import jax
import jax.numpy as jnp
from jax.experimental import pallas as pl

BATCH = 16384
NUM_FEATURES = 26
EMB_DIM = 64


def _tc_body(x_ref, w_ref, out_ref, norm_ref):
    w = w_ref[...]
    e = jnp.exp(w)
    s = jnp.sum(e, axis=0, keepdims=True)
    n = e / s
    norm_ref[...] = n
    x = x_ref[...]
    out_ref[...] = jnp.sum(x * n[:, None, :], axis=0)


def _tc_pool(xt, gathered_t, block_b=1024):
    nb = BATCH // block_b
    out_shapes = (
        jax.ShapeDtypeStruct((EMB_DIM, BATCH), jnp.float32),
        jax.ShapeDtypeStruct((NUM_FEATURES, BATCH), jnp.float32),
    )
    return pl.pallas_call(
        _tc_body,
        grid=(nb,),
        in_specs=[
            pl.BlockSpec((NUM_FEATURES, EMB_DIM, block_b), lambda i: (0, 0, i)),
            pl.BlockSpec((NUM_FEATURES, block_b), lambda i: (0, i)),
        ],
        out_specs=(
            pl.BlockSpec((EMB_DIM, block_b), lambda i: (0, i)),
            pl.BlockSpec((NUM_FEATURES, block_b), lambda i: (0, i)),
        ),
        out_shape=out_shapes,
    )(xt, gathered_t)


@jax.jit
def kernel(inputs, item_indices, attention_weights):
    xt = jnp.transpose(inputs, (1, 2, 0))
    g = jax.lax.gather(
        attention_weights, item_indices[:, None],
        jax.lax.GatherDimensionNumbers(
            offset_dims=(1,), collapsed_slice_dims=(0,), start_index_map=(0,)),
        slice_sizes=(1, NUM_FEATURES),
        mode=jax.lax.GatherScatterMode.PROMISE_IN_BOUNDS)
    out_t, norm_t = _tc_pool(xt, g.T)
    return out_t.T, norm_t.T[:, :, None]
